# baseline (speedup 1.0000x reference)
import functools
import math

import jax
import jax.numpy as jnp
from jax import lax
from jax.experimental import pallas as pl
from jax.experimental.pallas import tpu as pltpu
from jax.experimental.pallas import tpu_sc as plsc

N = 100000
K = 16
E = N * K
TAU = 2.0
GAMMA = -0.1
ZETA = 1.1
EPS = 1e-06
_C = math.log((0.0 - GAMMA) / (ZETA - 0.0) + EPS)

BCE = 160000
GJ = E // BCE

NW = 32
PER_W = E // NW
CHUNK = 10000
REPS = PER_W // CHUNK


def _edges_kernel(logit_ref, edges_ref, pen_ref, s0_ref):
    j = pl.program_id(0)

    @pl.when(j == 0)
    def _():
        c = jax.lax.broadcasted_iota(jnp.int32, (2, BCE), 1)
        row = jax.lax.broadcasted_iota(jnp.int32, (2, BCE), 0)
        s0_ref[...] = (c >> 4) + jnp.where(row == 0, 0, (c & (K - 1)) + 1)
        pen_ref[0] = jax.nn.sigmoid(logit_ref[0] - TAU * _C)

    v = s0_ref[...] + j * (BCE // K)

    @pl.when(j < GJ - 1)
    def _():
        edges_ref[...] = v

    @pl.when(j == GJ - 1)
    def _():
        edges_ref[...] = jnp.where(v >= N, v - N, v)


def _weights_sc_kernel(logit_hbm, out_hbm, lv_ref, buf_ref):
    wid = lax.axis_index("s") * 2 + lax.axis_index("c")
    pltpu.sync_copy(logit_hbm, lv_ref)
    lv = lv_ref[...]
    s = 1.0 / (1.0 + jnp.exp(-lv / TAU))
    gate = jnp.minimum(jnp.maximum(s * (ZETA - GAMMA) + GAMMA, 0.0), 1.0)

    def fill(i, _):
        buf_ref[pl.ds(i * 16, 16)] = gate
        return _

    lax.fori_loop(0, CHUNK // 16, fill, 0)
    base = wid * PER_W
    for r in range(REPS):
        pltpu.sync_copy(buf_ref, out_hbm.at[pl.ds(base + r * CHUNK, CHUNK)])


_weights_sc = functools.partial(
    pl.kernel,
    out_type=jax.ShapeDtypeStruct((E,), jnp.float32),
    mesh=plsc.VectorSubcoreMesh(core_axis_name="c", subcore_axis_name="s"),
    scratch_types=[
        pltpu.VMEM((16,), jnp.float32),
        pltpu.VMEM((CHUNK,), jnp.float32),
    ],
)(_weights_sc_kernel)


def kernel(x, batch, logit):
    del x, batch
    edges, pen = pl.pallas_call(
        _edges_kernel,
        grid=(GJ,),
        in_specs=[pl.BlockSpec(memory_space=pltpu.SMEM)],
        out_specs=[
            pl.BlockSpec((2, BCE), lambda j: (0, j)),
            pl.BlockSpec(memory_space=pltpu.SMEM),
        ],
        out_shape=[
            jax.ShapeDtypeStruct((2, E), jnp.int32),
            jax.ShapeDtypeStruct((1,), jnp.float32),
        ],
        scratch_shapes=[pltpu.VMEM((2, BCE), jnp.int32)],
    )(logit)
    weights = _weights_sc(jnp.broadcast_to(logit, (16,)))
    return edges, weights, pen.reshape(())


# speedup vs baseline: 1.0620x; 1.0620x over previous
import functools
import math

import jax
import jax.numpy as jnp
from jax import lax
from jax.experimental import pallas as pl
from jax.experimental.pallas import tpu as pltpu
from jax.experimental.pallas import tpu_sc as plsc

N = 100000
K = 16
E = N * K
TAU = 2.0
GAMMA = -0.1
ZETA = 1.1
EPS = 1e-06
_C = math.log((0.0 - GAMMA) / (ZETA - 0.0) + EPS)

BCE = 160000
GJ = E // BCE

NW = 32
PER_W = E // NW
CHUNK = 2000
REPS = PER_W // CHUNK


def _edges_kernel(logit_ref, edges_ref, pen_ref, s0_ref):
    j = pl.program_id(0)

    @pl.when(j == 0)
    def _():
        c = jax.lax.broadcasted_iota(jnp.int32, (2, BCE), 1)
        row = jax.lax.broadcasted_iota(jnp.int32, (2, BCE), 0)
        s0_ref[...] = (c >> 4) + jnp.where(row == 0, 0, (c & (K - 1)) + 1)
        pen_ref[0] = jax.nn.sigmoid(logit_ref[0] - TAU * _C)

    v = s0_ref[...] + j * (BCE // K)

    @pl.when(j < GJ - 1)
    def _():
        edges_ref[...] = v

    @pl.when(j == GJ - 1)
    def _():
        edges_ref[...] = jnp.where(v >= N, v - N, v)


def _weights_sc_kernel(logit_hbm, out_hbm, lv_ref, buf_ref, sem):
    wid = lax.axis_index("s") * 2 + lax.axis_index("c")
    pltpu.sync_copy(logit_hbm, lv_ref)
    lv = lv_ref[...]
    s = 1.0 / (1.0 + jnp.exp(-lv / TAU))
    gate = jnp.minimum(jnp.maximum(s * (ZETA - GAMMA) + GAMMA, 0.0), 1.0)

    def fill(i, _):
        base = i * 80
        buf_ref[pl.ds(base, 16)] = gate
        buf_ref[pl.ds(base + 16, 16)] = gate
        buf_ref[pl.ds(base + 32, 16)] = gate
        buf_ref[pl.ds(base + 48, 16)] = gate
        buf_ref[pl.ds(base + 64, 16)] = gate
        return _

    lax.fori_loop(0, CHUNK // 80, fill, 0)
    base = wid * PER_W
    copies = [
        pltpu.async_copy(buf_ref, out_hbm.at[pl.ds(base + r * CHUNK, CHUNK)], sem)
        for r in range(REPS)
    ]
    for c in copies:
        c.wait()


_weights_sc = functools.partial(
    pl.kernel,
    out_type=jax.ShapeDtypeStruct((E,), jnp.float32),
    mesh=plsc.VectorSubcoreMesh(core_axis_name="c", subcore_axis_name="s"),
    scratch_types=[
        pltpu.VMEM((16,), jnp.float32),
        pltpu.VMEM((CHUNK,), jnp.float32),
        pltpu.SemaphoreType.DMA,
    ],
)(_weights_sc_kernel)


def kernel(x, batch, logit):
    del x, batch
    weights = _weights_sc(jnp.broadcast_to(logit, (16,)))
    edges, pen = pl.pallas_call(
        _edges_kernel,
        grid=(GJ,),
        in_specs=[pl.BlockSpec(memory_space=pltpu.SMEM)],
        out_specs=[
            pl.BlockSpec((2, BCE), lambda j: (0, j)),
            pl.BlockSpec(memory_space=pltpu.SMEM),
        ],
        out_shape=[
            jax.ShapeDtypeStruct((2, E), jnp.int32),
            jax.ShapeDtypeStruct((1,), jnp.float32),
        ],
        scratch_shapes=[pltpu.VMEM((2, BCE), jnp.int32)],
    )(logit)
    return edges, weights, pen.reshape(())


# speedup vs baseline: 2.7542x; 2.5935x over previous
import math

import jax
import jax.numpy as jnp
from jax.experimental import pallas as pl
from jax.experimental.pallas import tpu as pltpu

N = 100000
K = 16
E = N * K
TAU = 2.0
GAMMA = -0.1
ZETA = 1.1
EPS = 1e-06
_C = math.log((0.0 - GAMMA) / (ZETA - 0.0) + EPS)

BCE = 160000
GJ = E // BCE
BWT = 524288


def _gen_kernel(logit_ref, edges_ref, weights_ref, pen_ref, s0_ref):
    j = pl.program_id(0)
    logit = logit_ref[0]

    @pl.when(j == 0)
    def _():
        c = jax.lax.broadcasted_iota(jnp.int32, (2, BCE), 1)
        row = jax.lax.broadcasted_iota(jnp.int32, (2, BCE), 0)
        s0_ref[...] = (c >> 4) + jnp.where(row == 0, 0, (c & (K - 1)) + 1)
        pen_ref[0] = jax.nn.sigmoid(logit - TAU * _C)

    @pl.when((j >= 1) & (j < 5))
    def _():
        s = jax.nn.sigmoid(logit / TAU)
        gate = jnp.clip(s * (ZETA - GAMMA) + GAMMA, 0.0, 1.0)
        weights_ref[...] = jnp.full((BWT,), gate, dtype=jnp.float32)

    v = s0_ref[...] + j * (BCE // K)

    @pl.when(j < GJ - 1)
    def _():
        edges_ref[...] = v

    @pl.when(j == GJ - 1)
    def _():
        edges_ref[...] = jnp.where(v >= N, v - N, v)


def kernel(x, batch, logit):
    del x, batch
    edges, weights, pen = pl.pallas_call(
        _gen_kernel,
        grid=(GJ,),
        in_specs=[pl.BlockSpec(memory_space=pltpu.SMEM)],
        out_specs=[
            pl.BlockSpec((2, BCE), lambda j: (0, j)),
            pl.BlockSpec((BWT,), lambda j: (jnp.clip(j - 1, 0, 3),)),
            pl.BlockSpec(memory_space=pltpu.SMEM),
        ],
        out_shape=[
            jax.ShapeDtypeStruct((2, E), jnp.int32),
            jax.ShapeDtypeStruct((E,), jnp.float32),
            jax.ShapeDtypeStruct((1,), jnp.float32),
        ],
        scratch_shapes=[pltpu.VMEM((2, BCE), jnp.int32)],
    )(logit)
    return edges, weights, pen.reshape(())
